# baseline (speedup 1.0000x reference)
import functools

import jax
import jax.numpy as jnp
from jax import lax
from jax.experimental import pallas as pl
from jax.experimental.pallas import tpu as pltpu
from jax.experimental.pallas import tpu_sc as plsc

CHUNK = 128
K = 2
GROUP = K * CHUNK
NBUF = 4


@functools.cache
def _make_lookup(N, D):
    info = plsc.get_sparse_core_info()
    nw = info.num_cores * info.num_subcores
    b_per_w = N // nw
    n_chunks = b_per_w // CHUNK
    n_groups = n_chunks // K
    n_main = n_groups - NBUF
    assert n_main % NBUF == 0
    mesh = plsc.VectorSubcoreMesh(core_axis_name="c", subcore_axis_name="s")

    @functools.partial(
        pl.kernel,
        mesh=mesh,
        out_type=jax.ShapeDtypeStruct((N, D), jnp.float32),
        compiler_params=pltpu.CompilerParams(use_tc_tiling_on_sc=False),
        scratch_types=[
            pltpu.VMEM((n_chunks, CHUNK), jnp.int32),
            pltpu.VMEM((NBUF, GROUP, D), jnp.float32),
            [pltpu.SemaphoreType.DMA] * NBUF,
            [pltpu.SemaphoreType.DMA] * NBUF,
        ],
    )
    def lookup(idx_hbm, table_hbm, out_hbm, idx_v, bufs, gsems, osems):
        wid = lax.axis_index("s") * info.num_cores + lax.axis_index("c")
        base = wid * b_per_w
        pltpu.sync_copy(idx_hbm.at[pl.ds(wid * n_chunks, n_chunks)], idx_v)

        def fire_gathers(g, b):
            for j in range(K):
                pltpu.async_copy(
                    table_hbm.at[idx_v.at[g * K + j]],
                    bufs.at[b, pl.ds(j * CHUNK, CHUNK)],
                    gsems[b],
                )

        def drain_gathers(g, b):
            for j in range(K):
                pltpu.make_async_copy(
                    table_hbm.at[idx_v.at[g * K + j]],
                    bufs.at[b, pl.ds(j * CHUNK, CHUNK)],
                    gsems[b],
                ).wait()

        def fire_out(g, b):
            pltpu.async_copy(
                bufs.at[b], out_hbm.at[pl.ds(base + g * GROUP, GROUP)],
                osems[b],
            )

        def wait_out(g, b):
            pltpu.make_async_copy(
                bufs.at[b], out_hbm.at[pl.ds(base + g * GROUP, GROUP)],
                osems[b],
            ).wait()

        for b in range(NBUF):
            fire_gathers(b, b)

        def body(i, carry):
            for b in range(NBUF):
                t = i * NBUF + b
                drain_gathers(t, b)
                fire_out(t, b)
                wait_out(t, b)
                fire_gathers(t + NBUF, b)
            return carry

        lax.fori_loop(0, n_main // NBUF, body, 0)

        for b in range(NBUF):
            t = n_main + b
            drain_gathers(t, b)
            fire_out(t, b)
        for b in range(NBUF):
            wait_out(n_main + b, b)

    return lookup


def kernel(x, table):
    B, L = x.shape
    D = table.shape[1]
    idx = x.reshape(-1, CHUNK).astype(jnp.int32)
    out = _make_lookup(B * L, D)(idx, table)
    return out.reshape(B, L, D)
